# baseline (speedup 1.0000x reference)
import jax
import jax.numpy as jnp
from jax.experimental import pallas as pl
from jax.experimental.pallas import tpu as pltpu

_BM = 1024


def _add_kernel(gid_ref, vec_ref, tok_ref, out_ref):
    del gid_ref
    out_ref[...] = tok_ref[...] + vec_ref[0]


def kernel(tokens, group_id, group_id_vecs):
    b, s, d = tokens.shape
    rows = b * s
    tok2d = tokens.reshape(rows, d)
    gid = jnp.asarray(group_id, jnp.int32).reshape((1,))
    table3d = group_id_vecs.reshape(group_id_vecs.shape[0], 1, d)
    grid = (rows // _BM,)
    out = pl.pallas_call(
        _add_kernel,
        grid_spec=pltpu.PrefetchScalarGridSpec(
            num_scalar_prefetch=1,
            grid=grid,
            in_specs=[
                pl.BlockSpec((1, 1, d), lambda i, gid_ref: (gid_ref[0], 0, 0)),
                pl.BlockSpec((_BM, d), lambda i, gid_ref: (i, 0)),
            ],
            out_specs=pl.BlockSpec((_BM, d), lambda i, gid_ref: (i, 0)),
        ),
        out_shape=jax.ShapeDtypeStruct((rows, d), tokens.dtype),
        compiler_params=pltpu.CompilerParams(
            dimension_semantics=("arbitrary",),
        ),
    )(gid, table3d, tok2d)
    return out.reshape(b, s, d)


# speedup vs baseline: 1.0325x; 1.0325x over previous
import jax
import jax.numpy as jnp
from jax.experimental import pallas as pl
from jax.experimental.pallas import tpu as pltpu

_BM = 2048


def _add_kernel(gid_ref, vec_ref, tok_ref, out_ref):
    del gid_ref
    out_ref[...] = tok_ref[...] + vec_ref[0]


def kernel(tokens, group_id, group_id_vecs):
    b, s, d = tokens.shape
    rows = b * s
    tok2d = tokens.reshape(rows, d)
    gid = jnp.asarray(group_id, jnp.int32).reshape((1,))
    table3d = group_id_vecs.reshape(group_id_vecs.shape[0], 1, d)
    grid = (rows // _BM,)
    out = pl.pallas_call(
        _add_kernel,
        grid_spec=pltpu.PrefetchScalarGridSpec(
            num_scalar_prefetch=1,
            grid=grid,
            in_specs=[
                pl.BlockSpec((1, 1, d), lambda i, gid_ref: (gid_ref[0], 0, 0)),
                pl.BlockSpec((_BM, d), lambda i, gid_ref: (i, 0)),
            ],
            out_specs=pl.BlockSpec((_BM, d), lambda i, gid_ref: (i, 0)),
        ),
        out_shape=jax.ShapeDtypeStruct((rows, d), tokens.dtype),
        compiler_params=pltpu.CompilerParams(
            dimension_semantics=("parallel",),
        ),
    )(gid, table3d, tok2d)
    return out.reshape(b, s, d)
